# Initial kernel scaffold; baseline (speedup 1.0000x reference)
import jax
import jax.numpy as jnp
from jax.experimental import pallas as pl


def kernel(amino_seq, struct_seq, amino_table, struct_table, pos_table, gamma, beta):
    raise NotImplementedError("write your pallas kernel here")


# speedup vs baseline: 11.4250x; 11.4250x over previous
import functools

import jax
import jax.numpy as jnp
from jax import lax
from jax.experimental import pallas as pl
from jax.experimental.pallas import tpu as pltpu
from jax.experimental.pallas import tpu_sc as plsc


def _build_table(pos, amino, struct, gamma, beta):
    L, D = pos.shape
    NA = amino.shape[0]
    NS = struct.shape[0]
    LB = 40

    def body(pos_ref, amino_ref, struct_ref, g_ref, b_ref, out_ref):
        a = amino_ref[...][None, :, None, :]
        s = struct_ref[...][None, None, :, :]
        p = pos_ref[...][:, None, None, :]
        x = a + s + p
        mean = jnp.mean(x, axis=-1, keepdims=True)
        var = jnp.mean((x - mean) ** 2, axis=-1, keepdims=True)
        y = (x - mean) * lax.rsqrt(var + 1e-5)
        y = y * g_ref[...][None, None, :] + b_ref[...][None, None, :]
        out_ref[...] = y

    return pl.pallas_call(
        body,
        grid=(L // LB,),
        in_specs=[
            pl.BlockSpec((LB, D), lambda i: (i, 0)),
            pl.BlockSpec((NA, D), lambda i: (0, 0)),
            pl.BlockSpec((NS, D), lambda i: (0, 0)),
            pl.BlockSpec((1, D), lambda i: (0, 0)),
            pl.BlockSpec((1, D), lambda i: (0, 0)),
        ],
        out_specs=pl.BlockSpec((LB, NA, NS, D), lambda i: (i, 0, 0, 0)),
        out_shape=jax.ShapeDtypeStruct((L, NA, NS, D), jnp.float32),
    )(pos, amino, struct, gamma.reshape(1, D), beta.reshape(1, D))


def _sc_lookup(a_idx, s_idx, table, seq_len, n_struct):
    info = plsc.get_sparse_core_info()
    nc, nsub, lanes = info.num_cores, info.num_subcores, info.num_lanes
    nw = nc * nsub
    T = a_idx.shape[0]
    D = table.shape[1]
    ncombo = table.shape[0] // seq_len
    per_w = T // nw
    CH = 128
    n_it = per_w // CH

    @functools.partial(
        pl.kernel,
        mesh=plsc.VectorSubcoreMesh(core_axis_name="c", subcore_axis_name="s"),
        out_type=jax.ShapeDtypeStruct((T, D), jnp.float32),
        scratch_types=[
            pltpu.VMEM((CH,), jnp.int32),
            pltpu.VMEM((CH,), jnp.int32),
            pltpu.VMEM((CH,), jnp.int32),
            pltpu.VMEM((CH, D), jnp.float32),
            pltpu.SemaphoreType.DMA,
        ],
    )
    def k(a_hbm, s_hbm, tbl_hbm, out_hbm, av, sv, fv, stage, sem):
        wid = lax.axis_index("s") * nc + lax.axis_index("c")
        w_base = wid * per_w

        def body(i, carry):
            base = pl.multiple_of(w_base + i * CH, 8)
            pltpu.sync_copy(a_hbm.at[pl.ds(base, CH)], av)
            pltpu.sync_copy(s_hbm.at[pl.ds(base, CH)], sv)
            for g in range(CH // lanes):
                va = av[pl.ds(g * lanes, lanes)]
                vs = sv[pl.ds(g * lanes, lanes)]
                vt = lax.iota(jnp.int32, lanes) + (base + g * lanes)
                vl = lax.rem(vt, seq_len)
                fv[pl.ds(g * lanes, lanes)] = vl * ncombo + va * n_struct + vs
            pltpu.async_copy(tbl_hbm.at[fv], stage, sem).wait()
            pltpu.sync_copy(stage, out_hbm.at[pl.ds(base, CH)])
            return carry

        lax.fori_loop(0, n_it, body, 0)

    return k(a_idx, s_idx, table)


def kernel(amino_seq, struct_seq, amino_table, struct_table, pos_table, gamma, beta):
    B, L = amino_seq.shape
    NA, D = amino_table.shape
    NS = struct_table.shape[0]
    table = _build_table(pos_table[:L], amino_table, struct_table, gamma, beta)
    table = table.reshape(L * NA * NS, D)
    flat = _sc_lookup(
        amino_seq.reshape(-1), struct_seq.reshape(-1), table, L, NS
    )
    return flat.reshape(B, L, D)


# speedup vs baseline: 19.4706x; 1.7042x over previous
import functools

import jax
import jax.numpy as jnp
from jax import lax
from jax.experimental import pallas as pl
from jax.experimental.pallas import tpu as pltpu
from jax.experimental.pallas import tpu_sc as plsc


def _build_table(pos, amino, struct, gamma, beta):
    L, D = pos.shape
    NA = amino.shape[0]
    NS = struct.shape[0]
    LB = 40

    def body(pos_ref, amino_ref, struct_ref, g_ref, b_ref, out_ref):
        a = amino_ref[...][None, :, None, :]
        s = struct_ref[...][None, None, :, :]
        p = pos_ref[...][:, None, None, :]
        x = a + s + p
        mean = jnp.mean(x, axis=-1, keepdims=True)
        var = jnp.mean((x - mean) ** 2, axis=-1, keepdims=True)
        y = (x - mean) * lax.rsqrt(var + 1e-5)
        y = y * g_ref[...][None, None, :] + b_ref[...][None, None, :]
        out_ref[...] = y

    return pl.pallas_call(
        body,
        grid=(L // LB,),
        in_specs=[
            pl.BlockSpec((LB, D), lambda i: (i, 0)),
            pl.BlockSpec((NA, D), lambda i: (0, 0)),
            pl.BlockSpec((NS, D), lambda i: (0, 0)),
            pl.BlockSpec((1, D), lambda i: (0, 0)),
            pl.BlockSpec((1, D), lambda i: (0, 0)),
        ],
        out_specs=pl.BlockSpec((LB, NA, NS, D), lambda i: (i, 0, 0, 0)),
        out_shape=jax.ShapeDtypeStruct((L, NA, NS, D), jnp.float32),
    )(pos, amino, struct, gamma.reshape(1, D), beta.reshape(1, D))


def _sc_lookup(a_idx, s_idx, table, seq_len, n_struct):
    info = plsc.get_sparse_core_info()
    nc, nsub, lanes = info.num_cores, info.num_subcores, info.num_lanes
    nw = nc * nsub
    T = a_idx.shape[0]
    D = table.shape[1]
    ncombo = table.shape[0] // seq_len
    per_w = T // nw
    CH = 128
    n_it = per_w // CH

    assert n_it % 2 == 0 and n_it >= 4

    @functools.partial(
        pl.kernel,
        mesh=plsc.VectorSubcoreMesh(core_axis_name="c", subcore_axis_name="s"),
        out_type=jax.ShapeDtypeStruct((T, D), jnp.float32),
        scratch_types=[
            pltpu.VMEM((2, CH), jnp.int32),
            pltpu.VMEM((2, CH), jnp.int32),
            pltpu.VMEM((2, CH), jnp.int32),
            pltpu.VMEM((2, CH, D), jnp.float32),
            pltpu.SemaphoreType.DMA((2,)),
            pltpu.SemaphoreType.DMA((2,)),
            pltpu.SemaphoreType.DMA((2,)),
        ],
    )
    def k(a_hbm, s_hbm, tbl_hbm, out_hbm, av, sv, fv, stage, isem, gsem, osem):
        wid = lax.axis_index("s") * nc + lax.axis_index("c")
        w_base = wid * per_w

        def idx_start(i, b):
            base = pl.multiple_of(w_base + i * CH, 8)
            pltpu.async_copy(a_hbm.at[pl.ds(base, CH)], av.at[b], isem.at[b])
            pltpu.async_copy(s_hbm.at[pl.ds(base, CH)], sv.at[b], isem.at[b])

        def idx_wait(i, b):
            base = pl.multiple_of(w_base + i * CH, 8)
            pltpu.make_async_copy(a_hbm.at[pl.ds(base, CH)], av.at[b], isem.at[b]).wait()
            pltpu.make_async_copy(s_hbm.at[pl.ds(base, CH)], sv.at[b], isem.at[b]).wait()

        def compute_fidx(i, b):
            base = w_base + i * CH
            for g in range(CH // lanes):
                va = av[b, pl.ds(g * lanes, lanes)]
                vs = sv[b, pl.ds(g * lanes, lanes)]
                vt = lax.iota(jnp.int32, lanes) + (base + g * lanes)
                vl = lax.rem(vt, seq_len)
                fv[b, pl.ds(g * lanes, lanes)] = vl * ncombo + va * n_struct + vs

        def gather_start(b):
            pltpu.async_copy(tbl_hbm.at[fv.at[b]], stage.at[b], gsem.at[b])

        def gather_wait(b):
            pltpu.make_async_copy(tbl_hbm.at[fv.at[b]], stage.at[b], gsem.at[b]).wait()

        def out_start(i, b):
            base = pl.multiple_of(w_base + i * CH, 8)
            pltpu.async_copy(stage.at[b], out_hbm.at[pl.ds(base, CH)], osem.at[b])

        def out_wait(i, b):
            base = pl.multiple_of(w_base + i * CH, 8)
            pltpu.make_async_copy(stage.at[b], out_hbm.at[pl.ds(base, CH)], osem.at[b]).wait()

        idx_start(0, 0)
        idx_start(1, 1)
        idx_wait(0, 0)
        compute_fidx(0, 0)
        gather_start(0)
        idx_wait(1, 1)
        compute_fidx(1, 1)
        gather_start(1)

        def body(kk, carry):
            i0 = kk * 2
            for b in range(2):
                i = i0 + b
                idx_start(i + 2, b)
                gather_wait(b)
                out_start(i, b)
                idx_wait(i + 2, b)
                compute_fidx(i + 2, b)
                out_wait(i, b)
                gather_start(b)
            return carry

        lax.fori_loop(0, n_it // 2 - 1, body, 0)

        gather_wait(0)
        out_start(n_it - 2, 0)
        gather_wait(1)
        out_start(n_it - 1, 1)
        out_wait(n_it - 2, 0)
        out_wait(n_it - 1, 1)

    return k(a_idx, s_idx, table)


def kernel(amino_seq, struct_seq, amino_table, struct_table, pos_table, gamma, beta):
    B, L = amino_seq.shape
    NA, D = amino_table.shape
    NS = struct_table.shape[0]
    table = _build_table(pos_table[:L], amino_table, struct_table, gamma, beta)
    table = table.reshape(L * NA * NS, D)
    flat = _sc_lookup(
        amino_seq.reshape(-1), struct_seq.reshape(-1), table, L, NS
    )
    return flat.reshape(B, L, D)


# speedup vs baseline: 19.6396x; 1.0087x over previous
import functools

import jax
import jax.numpy as jnp
from jax import lax
from jax.experimental import pallas as pl
from jax.experimental.pallas import tpu as pltpu
from jax.experimental.pallas import tpu_sc as plsc


def _build_table(pos, amino, struct, gamma, beta):
    L, D = pos.shape
    NA = amino.shape[0]
    NS = struct.shape[0]
    LB = 40

    def body(pos_ref, amino_ref, struct_ref, g_ref, b_ref, out_ref):
        a = amino_ref[...][None, :, None, :]
        s = struct_ref[...][None, None, :, :]
        p = pos_ref[...][:, None, None, :]
        x = a + s + p
        mean = jnp.mean(x, axis=-1, keepdims=True)
        var = jnp.mean((x - mean) ** 2, axis=-1, keepdims=True)
        y = (x - mean) * lax.rsqrt(var + 1e-5)
        y = y * g_ref[...][None, None, :] + b_ref[...][None, None, :]
        out_ref[...] = y

    return pl.pallas_call(
        body,
        grid=(L // LB,),
        in_specs=[
            pl.BlockSpec((LB, D), lambda i: (i, 0)),
            pl.BlockSpec((NA, D), lambda i: (0, 0)),
            pl.BlockSpec((NS, D), lambda i: (0, 0)),
            pl.BlockSpec((1, D), lambda i: (0, 0)),
            pl.BlockSpec((1, D), lambda i: (0, 0)),
        ],
        out_specs=pl.BlockSpec((LB, NA, NS, D), lambda i: (i, 0, 0, 0)),
        out_shape=jax.ShapeDtypeStruct((L, NA, NS, D), jnp.float32),
    )(pos, amino, struct, gamma.reshape(1, D), beta.reshape(1, D))


def _sc_lookup(a_idx, s_idx, table, seq_len, n_struct):
    info = plsc.get_sparse_core_info()
    nc, nsub, lanes = info.num_cores, info.num_subcores, info.num_lanes
    nw = nc * nsub
    T = a_idx.shape[0]
    D = table.shape[1]
    ncombo = table.shape[0] // seq_len
    per_w = T // nw
    CH = 128
    n_it = per_w // CH

    NB = 5
    assert n_it % NB == 0 and n_it // NB >= 2

    @functools.partial(
        pl.kernel,
        mesh=plsc.VectorSubcoreMesh(core_axis_name="c", subcore_axis_name="s"),
        out_type=jax.ShapeDtypeStruct((T, D), jnp.float32),
        scratch_types=[
            pltpu.VMEM((NB, CH), jnp.int32),
            pltpu.VMEM((NB, CH), jnp.int32),
            pltpu.VMEM((NB, CH), jnp.int32),
            pltpu.VMEM((NB, CH, D), jnp.float32),
            pltpu.SemaphoreType.DMA((NB,)),
            pltpu.SemaphoreType.DMA((NB,)),
            pltpu.SemaphoreType.DMA((NB,)),
        ],
    )
    def k(a_hbm, s_hbm, tbl_hbm, out_hbm, av, sv, fv, stage, isem, gsem, osem):
        wid = lax.axis_index("s") * nc + lax.axis_index("c")
        w_base = wid * per_w

        def idx_start(i, b):
            base = pl.multiple_of(w_base + i * CH, 8)
            pltpu.async_copy(a_hbm.at[pl.ds(base, CH)], av.at[b], isem.at[b])
            pltpu.async_copy(s_hbm.at[pl.ds(base, CH)], sv.at[b], isem.at[b])

        def idx_wait(i, b):
            base = pl.multiple_of(w_base + i * CH, 8)
            pltpu.make_async_copy(a_hbm.at[pl.ds(base, CH)], av.at[b], isem.at[b]).wait()
            pltpu.make_async_copy(s_hbm.at[pl.ds(base, CH)], sv.at[b], isem.at[b]).wait()

        def compute_fidx(i, b):
            base = w_base + i * CH
            for g in range(CH // lanes):
                va = av[b, pl.ds(g * lanes, lanes)]
                vs = sv[b, pl.ds(g * lanes, lanes)]
                vt = lax.iota(jnp.int32, lanes) + (base + g * lanes)
                vl = lax.rem(vt, seq_len)
                fv[b, pl.ds(g * lanes, lanes)] = vl * ncombo + va * n_struct + vs

        def gather_start(b):
            pltpu.async_copy(tbl_hbm.at[fv.at[b]], stage.at[b], gsem.at[b])

        def gather_wait(b):
            pltpu.make_async_copy(tbl_hbm.at[fv.at[b]], stage.at[b], gsem.at[b]).wait()

        def out_start(i, b):
            base = pl.multiple_of(w_base + i * CH, 8)
            pltpu.async_copy(stage.at[b], out_hbm.at[pl.ds(base, CH)], osem.at[b])

        def out_wait(i, b):
            base = pl.multiple_of(w_base + i * CH, 8)
            pltpu.make_async_copy(stage.at[b], out_hbm.at[pl.ds(base, CH)], osem.at[b]).wait()

        for b in range(NB):
            idx_start(b, b)
        for b in range(NB):
            idx_wait(b, b)
            compute_fidx(b, b)
            gather_start(b)

        def body(kk, carry):
            i0 = kk * NB
            for b in range(NB):
                i = i0 + b
                idx_start(i + NB, b)
                gather_wait(b)
                out_start(i, b)
                idx_wait(i + NB, b)
                compute_fidx(i + NB, b)
                out_wait(i, b)
                gather_start(b)
            return carry

        lax.fori_loop(0, n_it // NB - 1, body, 0)

        for b in range(NB):
            gather_wait(b)
            out_start(n_it - NB + b, b)
        for b in range(NB):
            out_wait(n_it - NB + b, b)

    return k(a_idx, s_idx, table)


def kernel(amino_seq, struct_seq, amino_table, struct_table, pos_table, gamma, beta):
    B, L = amino_seq.shape
    NA, D = amino_table.shape
    NS = struct_table.shape[0]
    table = _build_table(pos_table[:L], amino_table, struct_table, gamma, beta)
    table = table.reshape(L * NA * NS, D)
    flat = _sc_lookup(
        amino_seq.reshape(-1), struct_seq.reshape(-1), table, L, NS
    )
    return flat.reshape(B, L, D)


# speedup vs baseline: 19.9707x; 1.0169x over previous
import functools

import jax
import jax.numpy as jnp
from jax import lax
from jax.experimental import pallas as pl
from jax.experimental.pallas import tpu as pltpu
from jax.experimental.pallas import tpu_sc as plsc


def _build_table(pos_full, amino, struct, gamma, beta, seq_len):
    D = pos_full.shape[1]
    NA = amino.shape[0]
    NS = struct.shape[0]
    LB = 40

    def body(pos_ref, amino_ref, struct_ref, g_ref, b_ref, out_ref):
        a = amino_ref[...][None, :, None, :]
        s = struct_ref[...][None, None, :, :]
        p = pos_ref[...][:, None, None, :]
        x = a + s + p
        mean = jnp.mean(x, axis=-1, keepdims=True)
        var = jnp.mean((x - mean) ** 2, axis=-1, keepdims=True)
        y = (x - mean) * lax.rsqrt(var + 1e-5)
        y = y * g_ref[...][None, None, :] + b_ref[...][None, None, :]
        out_ref[...] = y

    return pl.pallas_call(
        body,
        grid=(seq_len // LB,),
        in_specs=[
            pl.BlockSpec((LB, D), lambda i: (i, 0)),
            pl.BlockSpec((NA, D), lambda i: (0, 0)),
            pl.BlockSpec((NS, D), lambda i: (0, 0)),
            pl.BlockSpec((1, D), lambda i: (0, 0)),
            pl.BlockSpec((1, D), lambda i: (0, 0)),
        ],
        out_specs=pl.BlockSpec((LB, NA, NS, D), lambda i: (i, 0, 0, 0)),
        out_shape=jax.ShapeDtypeStruct((seq_len, NA, NS, D), jnp.float32),
    )(pos_full, amino, struct, gamma.reshape(1, D), beta.reshape(1, D))


def _sc_lookup(a_idx, s_idx, table, seq_len, n_struct):
    info = plsc.get_sparse_core_info()
    nc, nsub, lanes = info.num_cores, info.num_subcores, info.num_lanes
    nw = nc * nsub
    T = a_idx.shape[0]
    D = table.shape[1]
    ncombo = table.shape[0] // seq_len
    per_w = T // nw
    CH = 128
    n_it = per_w // CH

    NB = 5
    assert n_it % NB == 0 and n_it // NB >= 2

    @functools.partial(
        pl.kernel,
        mesh=plsc.VectorSubcoreMesh(core_axis_name="c", subcore_axis_name="s"),
        out_type=jax.ShapeDtypeStruct((T, D), jnp.float32),
        scratch_types=[
            pltpu.VMEM((NB, CH), jnp.int32),
            pltpu.VMEM((NB, CH), jnp.int32),
            pltpu.VMEM((NB, CH), jnp.int32),
            pltpu.VMEM((NB, CH, D), jnp.float32),
            pltpu.SemaphoreType.DMA((NB,)),
            pltpu.SemaphoreType.DMA((NB,)),
            pltpu.SemaphoreType.DMA((NB,)),
        ],
    )
    def k(a_hbm, s_hbm, tbl_hbm, out_hbm, av, sv, fv, stage, isem, gsem, osem):
        sid = lax.axis_index("s")
        wid = sid * nc + lax.axis_index("c")
        w_base = wid * per_w

        def idx_start(i, b):
            base = pl.multiple_of(w_base + i * CH, 8)
            pltpu.async_copy(a_hbm.at[pl.ds(base, CH)], av.at[b], isem.at[b])
            pltpu.async_copy(s_hbm.at[pl.ds(base, CH)], sv.at[b], isem.at[b])

        def idx_wait(i, b):
            base = pl.multiple_of(w_base + i * CH, 8)
            pltpu.make_async_copy(a_hbm.at[pl.ds(base, CH)], av.at[b], isem.at[b]).wait()
            pltpu.make_async_copy(s_hbm.at[pl.ds(base, CH)], sv.at[b], isem.at[b]).wait()

        def compute_fidx(i, b):
            base = w_base + i * CH
            for g in range(CH // lanes):
                va = av[b, pl.ds(g * lanes, lanes)]
                vs = sv[b, pl.ds(g * lanes, lanes)]
                vt = lax.iota(jnp.int32, lanes) + (base + g * lanes)
                vl = lax.rem(vt, seq_len)
                fv[b, pl.ds(g * lanes, lanes)] = vl * ncombo + va * n_struct + vs

        def gather_start(b):
            pltpu.async_copy(tbl_hbm.at[fv.at[b]], stage.at[b], gsem.at[b])

        def gather_wait(b):
            pltpu.make_async_copy(tbl_hbm.at[fv.at[b]], stage.at[b], gsem.at[b]).wait()

        def out_start(i, b):
            base = pl.multiple_of(w_base + i * CH, 8)
            pltpu.async_copy(stage.at[b], out_hbm.at[pl.ds(base, CH)], osem.at[b])

        def out_wait(i, b):
            base = pl.multiple_of(w_base + i * CH, 8)
            pltpu.make_async_copy(stage.at[b], out_hbm.at[pl.ds(base, CH)], osem.at[b]).wait()

        for b in range(NB):
            idx_start(b, b)
        for b in range(NB):
            idx_wait(b, b)
            compute_fidx(b, b)
            gather_start(b)

        def body(kk, carry):
            i0 = kk * NB
            for b in range(NB):
                i = i0 + b
                idx_start(i + NB, b)
                gather_wait(b)
                out_start(i, b)
                idx_wait(i + NB, b)
                compute_fidx(i + NB, b)
                out_wait(i, b)
                gather_start(b)
            return carry

        lax.fori_loop(0, n_it // NB - 1, body, 0)

        for b in range(NB):
            gather_wait(b)
            out_start(n_it - NB + b, b)
        for b in range(NB):
            out_wait(n_it - NB + b, b)

    return k(a_idx, s_idx, table)


def kernel(amino_seq, struct_seq, amino_table, struct_table, pos_table, gamma, beta):
    B, L = amino_seq.shape
    NA, D = amino_table.shape
    NS = struct_table.shape[0]
    table = _build_table(pos_table, amino_table, struct_table, gamma, beta, L)
    table = table.reshape(L * NA * NS, D)
    flat = _sc_lookup(
        amino_seq.reshape(-1), struct_seq.reshape(-1), table, L, NS
    )
    return flat.reshape(B, L, D)
